# baseline (speedup 1.0000x reference)
import functools

import jax
import jax.numpy as jnp
from jax import lax
from jax.experimental import pallas as pl
from jax.experimental.pallas import tpu as pltpu
from jax.experimental.pallas import tpu_sc as plsc

N_NODES = 10000
N_EDGES = 320000
IN_SIZE = 128
OUT_SIZE = 16
NUM_HEADS = 8
HH = NUM_HEADS // 2
HC = OUT_SIZE * HH

NS = 16
EPT = N_EDGES // NS
K = 400
NCH = EPT // K
G = K // 16
K2 = 160
NCH2 = EPT // K2
G2 = K2 // 16
RPT = 640
SUB = 80


def _leaky(x):
    return jnp.maximum(x, 0.01 * x)


def _prep_body(z_ref, wt_ref, b_ref, al_ref, ar_ref,
               zp_ref, el_ref, er_ref, m_ref):
    z = z_ref[...]
    zp = jnp.dot(z, wt_ref[...], preferred_element_type=jnp.float32)
    zp = zp + b_ref[...][None, :]
    el = jnp.dot(zp, al_ref[...], preferred_element_type=jnp.float32)
    er = jnp.dot(zp, ar_ref[...], preferred_element_type=jnp.float32)
    gmax = jnp.max(el, axis=0, keepdims=True)
    zp_ref[...] = zp
    el_ref[...] = el
    er_ref[...] = er
    m_ref[...] = _leaky(gmax + er)


def _dense_prep(Z, W, b, a_l, a_r):
    n = Z.shape[0]
    nf = OUT_SIZE * NUM_HEADS
    jp = jnp.arange(nf)
    c = jp // HC
    f = (jp % HC) // HH
    hp = jp % HH
    perm = f * NUM_HEADS + c * HH + hp
    Wp = W[perm]
    bp = b[perm]
    h_of = c * HH + hp
    Al = jnp.zeros((nf, NUM_HEADS), jnp.float32).at[jp, h_of].set(a_l[0][f, h_of])
    Ar = jnp.zeros((nf, NUM_HEADS), jnp.float32).at[jp, h_of].set(a_r[0][f, h_of])
    return pl.pallas_call(
        _prep_body,
        out_shape=[
            jax.ShapeDtypeStruct((n, nf), jnp.float32),
            jax.ShapeDtypeStruct((n, NUM_HEADS), jnp.float32),
            jax.ShapeDtypeStruct((n, NUM_HEADS), jnp.float32),
            jax.ShapeDtypeStruct((n, NUM_HEADS), jnp.float32),
        ],
    )(Z, Wp.T, bp, Al, Ar)


_MESH = plsc.VectorSubcoreMesh(core_axis_name="c", subcore_axis_name="s")


@functools.partial(
    pl.kernel,
    out_type=[
        jax.ShapeDtypeStruct((2 * N_NODES, HC), jnp.float32),
        jax.ShapeDtypeStruct((2 * N_NODES, HC), jnp.float32),
        jax.ShapeDtypeStruct((2 * NS * NCH * K, 16), jnp.float32),
    ],
    mesh=_MESH,
    compiler_params=pltpu.CompilerParams(
        needs_layout_passes=False, use_tc_tiling_on_sc=False),
    scratch_types=[
        [pltpu.VMEM((K,), jnp.int32),
         pltpu.VMEM((K,), jnp.int32)],
        [pltpu.VMEM((K,), jnp.int32),
         pltpu.VMEM((K,), jnp.int32)],
        [pltpu.VMEM((K,), jnp.int32),
         pltpu.VMEM((K,), jnp.int32)],
        [pltpu.VMEM((K, 16), jnp.float32),
         pltpu.VMEM((K, 16), jnp.float32)],
        [pltpu.VMEM((K, 16), jnp.float32),
         pltpu.VMEM((K, 16), jnp.float32)],
        [pltpu.VMEM((K, 16), jnp.float32),
         pltpu.VMEM((K, 16), jnp.float32)],
        [pltpu.SemaphoreType.DMA,
         pltpu.SemaphoreType.DMA],
        [pltpu.SemaphoreType.DMA,
         pltpu.SemaphoreType.DMA],
        pltpu.VMEM((SUB, 16), jnp.float32),
        [pltpu.VMEM((K2,), jnp.int32),
         pltpu.VMEM((K2,), jnp.int32)],
        [pltpu.VMEM((K2,), jnp.int32),
         pltpu.VMEM((K2,), jnp.int32)],
        [pltpu.VMEM((K2, 16), jnp.float32),
         pltpu.VMEM((K2, 16), jnp.float32)],
        [pltpu.VMEM((K2, HC), jnp.float32),
         pltpu.VMEM((K2, HC), jnp.float32)],
        [pltpu.SemaphoreType.DMA,
         pltpu.SemaphoreType.DMA],
        [pltpu.SemaphoreType.DMA,
         pltpu.SemaphoreType.DMA],
        pltpu.VMEM_SHARED((N_NODES, 16), jnp.float32),
        pltpu.VMEM_SHARED((N_NODES, HC), jnp.float32),
    ],
)
def _sc_edges(idx_cat, tl2, trm2, zp2, rout, qbuf, tbuf,
              i_sq, i_dst, i_dq, tl_b, trm_b, tq, semg, semh, s_b,
              i_s2, i_dq2, tq2, rows2, semq, semt,
              S_sh, R_sh):
    cid = lax.axis_index("c")
    sid = lax.axis_index("s")
    iota = lax.iota(jnp.int32, 16)
    zeros16 = jnp.zeros((16,), jnp.float32)

    nbase = sid * RPT
    nsub = jnp.where(sid < NS - 1, RPT // SUB,
                     (N_NODES - (NS - 1) * RPT) // SUB)

    tile_s = sid * EPT
    tile_d = N_EDGES + sid * EPT
    tile_t = (cid * NS + sid) * EPT

    rows0 = rows2[0]

    def _zero_rows(i, _):
        for j in range(HC // 16):
            rows0[i, pl.ds(j * 16, 16)] = zeros16
        s_b[i, :] = zeros16
        return 0
    lax.fori_loop(0, SUB, _zero_rows, 0)

    def _zero_stripe(s, _):
        off = nbase + s * SUB
        pltpu.sync_copy(rows0.at[pl.ds(0, SUB)], R_sh.at[pl.ds(off, SUB)])
        pltpu.sync_copy(s_b, S_sh.at[pl.ds(off, SUB)])
        return 0
    lax.fori_loop(0, nsub, _zero_stripe, 0)
    plsc.subcore_barrier()

    def _p1_stage(ch, sl):
        pltpu.sync_copy(idx_cat.at[pl.ds(tile_s + ch * K, K)], i_sq[sl])
        pltpu.sync_copy(idx_cat.at[pl.ds(tile_d + ch * K, K)], i_dst[sl])
        for g in range(G):
            sv = i_sq[sl][pl.ds(g * 16, 16)]
            dv = i_dst[sl][pl.ds(g * 16, 16)]
            i_sq[sl][pl.ds(g * 16, 16)] = sv + cid * N_NODES
            i_dq[sl][pl.ds(g * 16, 16)] = dv + cid * N_NODES
        pltpu.async_copy(tl2.at[i_sq[sl]], tl_b[sl], semg[sl])
        pltpu.async_copy(trm2.at[i_dq[sl]], trm_b[sl], semh[sl])

    def _p1_finish(ch, sl):
        pltpu.make_async_copy(tl2.at[pl.ds(0, K)], tl_b[sl], semg[sl]).wait()
        pltpu.make_async_copy(trm2.at[pl.ds(0, K)], trm_b[sl], semh[sl]).wait()
        for g in range(G):
            ri = iota + g * 16
            for h in range(HH):
                hc = jnp.full((16,), h, jnp.int32)
                el = plsc.load_gather(tl_b[sl], [ri, hc])
                er = plsc.load_gather(trm_b[sl], [ri, hc])
                m = plsc.load_gather(
                    trm_b[sl], [ri, jnp.full((16,), HH + h, jnp.int32)])
                t = jnp.exp(_leaky(el + er) - m)
                for j in range(4):
                    plsc.store_scatter(
                        tq[sl], [ri, jnp.full((16,), j * HH + h, jnp.int32)], t)
        pltpu.sync_copy(tq[sl], S_sh.at[i_dst[sl]], add=True)
        pltpu.sync_copy(tq[sl], tbuf.at[pl.ds(tile_t + ch * K, K)])

    _p1_stage(0, 0)

    def _p1_pair(p, _):
        _p1_stage(2 * p + 1, 1)
        _p1_finish(2 * p, 0)

        @pl.when(p < NCH // 2 - 1)
        def _():
            _p1_stage(2 * p + 2, 0)
        _p1_finish(2 * p + 1, 1)
        return 0
    lax.fori_loop(0, NCH // 2, _p1_pair, 0)
    plsc.subcore_barrier()

    def _q_sub(s, _):
        off = nbase + s * SUB
        pltpu.sync_copy(zp2.at[pl.ds(cid * N_NODES + off, SUB)],
                        rows0.at[pl.ds(0, SUB)])
        pltpu.sync_copy(S_sh.at[pl.ds(off, SUB)], s_b)
        for i in range(SUB):
            sq = 1.0 / (s_b[i, :] + 1e-16)
            for s16 in range(HC // 16):
                v = rows0[i, pl.ds(s16 * 16, 16)]
                rows0[i, pl.ds(s16 * 16, 16)] = v * sq
        pltpu.sync_copy(rows0.at[pl.ds(0, SUB)],
                        qbuf.at[pl.ds(cid * N_NODES + off, SUB)])
        return 0
    lax.fori_loop(0, nsub, _q_sub, 0)
    plsc.subcore_barrier()

    def _p2_issue(ch, sl):
        pltpu.sync_copy(idx_cat.at[pl.ds(tile_s + ch * K2, K2)], i_s2[sl])
        pltpu.sync_copy(idx_cat.at[pl.ds(tile_d + ch * K2, K2)], i_dq2[sl])
        for g in range(G2):
            dv = i_dq2[sl][pl.ds(g * 16, 16)]
            i_dq2[sl][pl.ds(g * 16, 16)] = dv + cid * N_NODES
        pltpu.async_copy(tbuf.at[pl.ds(tile_t + ch * K2, K2)], tq2[sl], semt[sl])
        pltpu.async_copy(qbuf.at[i_dq2[sl]], rows2[sl], semq[sl])

    def _p2_finish(ch, sl):
        pltpu.make_async_copy(tbuf.at[pl.ds(tile_t + ch * K2, K2)], tq2[sl],
                              semt[sl]).wait()
        pltpu.make_async_copy(qbuf.at[pl.ds(0, K2)], rows2[sl], semq[sl]).wait()

        def _scale_grp(g, _):
            for j in range(16):
                i = g * 16 + j
                tv = tq2[sl][i, :]
                for s16 in range(HC // 16):
                    v = rows2[sl][i, pl.ds(s16 * 16, 16)]
                    rows2[sl][i, pl.ds(s16 * 16, 16)] = v * tv
            return 0
        lax.fori_loop(0, G2, _scale_grp, 0)
        pltpu.sync_copy(rows2[sl], R_sh.at[i_s2[sl]], add=True)

    _p2_issue(0, 0)

    def _p2_pair(p, _):
        _p2_issue(2 * p + 1, 1)
        _p2_finish(2 * p, 0)
        _p2_issue(2 * p + 2, 0)
        _p2_finish(2 * p + 1, 1)
        return 0
    lax.fori_loop(0, NCH2 // 2, _p2_pair, 0)
    _p2_finish(NCH2 - 1, 0)
    plsc.subcore_barrier()

    def _r_out(s, _):
        off = nbase + s * SUB
        pltpu.sync_copy(R_sh.at[pl.ds(off, SUB)], rows0.at[pl.ds(0, SUB)])
        pltpu.sync_copy(rows0.at[pl.ds(0, SUB)],
                        rout.at[pl.ds(cid * N_NODES + off, SUB)])
        return 0
    lax.fori_loop(0, nsub, _r_out, 0)


def kernel(index, n, Z, W, b, a_l, a_r):
    num_nodes = Z.shape[0]
    Zp, El, Er, M = _dense_prep(Z, W, b, a_l, a_r)
    idx_cat = index.astype(jnp.int32).reshape(-1)
    pad = jnp.zeros((2 * num_nodes, 2 * HH), jnp.float32)
    tl2 = jnp.concatenate(
        [jnp.concatenate([El[:, :HH], El[:, HH:]], axis=0),
         jnp.zeros((2 * num_nodes, 16 - HH), jnp.float32)], axis=1)
    trm2 = jnp.concatenate(
        [jnp.concatenate([Er[:, :HH], M[:, :HH]], axis=1),
         jnp.concatenate([Er[:, HH:], M[:, HH:]], axis=1)], axis=0)
    trm2 = jnp.concatenate([trm2, pad], axis=1)
    zp2 = jnp.concatenate([Zp[:, :HC], Zp[:, HC:]], axis=0)
    rout, _, _ = _sc_edges(idx_cat, tl2, trm2, zp2)
    rst = jnp.concatenate(
        [rout[:num_nodes].reshape(num_nodes, OUT_SIZE, HH),
         rout[num_nodes:].reshape(num_nodes, OUT_SIZE, HH)], axis=2)
    return rst * (jnp.asarray(n, dtype=rst.dtype) / num_nodes)


# speedup vs baseline: 1.0836x; 1.0836x over previous
import functools

import jax
import jax.numpy as jnp
from jax import lax
from jax.experimental import pallas as pl
from jax.experimental.pallas import tpu as pltpu
from jax.experimental.pallas import tpu_sc as plsc

N_NODES = 10000
N_EDGES = 320000
IN_SIZE = 128
OUT_SIZE = 16
NUM_HEADS = 8
HH = NUM_HEADS // 2
HC = OUT_SIZE * HH

NS = 16
EPT = N_EDGES // NS
K = 400
NCH = EPT // K
G = K // 16
K2 = 160
NCH2 = EPT // K2
G2 = K2 // 16
RPT = 640
SUB = 80


def _leaky(x):
    return jnp.maximum(x, 0.01 * x)


def _prep_body(z_ref, wt_ref, b_ref, al_ref, ar_ref,
               zp2_ref, tl2_ref, trm2_ref):
    n = z_ref.shape[0]
    z = z_ref[...]
    zp = jnp.dot(z, wt_ref[...], preferred_element_type=jnp.float32)
    zp = zp + b_ref[...][None, :]
    el = jnp.dot(zp, al_ref[...], preferred_element_type=jnp.float32)
    er = jnp.dot(zp, ar_ref[...], preferred_element_type=jnp.float32)
    gmax = jnp.max(el, axis=0, keepdims=True)
    m = _leaky(gmax + er)
    zeros12 = jnp.zeros((n, 16 - HH), jnp.float32)
    zp2_ref[0:n] = zp[:, :HC]
    zp2_ref[n:] = zp[:, HC:]
    tl2_ref[0:n] = jnp.concatenate([el[:, :HH], zeros12], axis=1)
    tl2_ref[n:] = jnp.concatenate([el[:, HH:], zeros12], axis=1)
    zeros8 = jnp.zeros((n, 2 * HH), jnp.float32)
    trm2_ref[0:n] = jnp.concatenate([er[:, :HH], m[:, :HH], zeros8], axis=1)
    trm2_ref[n:] = jnp.concatenate([er[:, HH:], m[:, HH:], zeros8], axis=1)


def _dense_prep(Z, W, b, a_l, a_r):
    n = Z.shape[0]
    nf = OUT_SIZE * NUM_HEADS
    jp = jnp.arange(nf)
    c = jp // HC
    f = (jp % HC) // HH
    hp = jp % HH
    perm = f * NUM_HEADS + c * HH + hp
    Wp = W[perm]
    bp = b[perm]
    h_of = c * HH + hp
    Al = jnp.zeros((nf, NUM_HEADS), jnp.float32).at[jp, h_of].set(a_l[0][f, h_of])
    Ar = jnp.zeros((nf, NUM_HEADS), jnp.float32).at[jp, h_of].set(a_r[0][f, h_of])
    return pl.pallas_call(
        _prep_body,
        out_shape=[
            jax.ShapeDtypeStruct((2 * n, HC), jnp.float32),
            jax.ShapeDtypeStruct((2 * n, 16), jnp.float32),
            jax.ShapeDtypeStruct((2 * n, 16), jnp.float32),
        ],
    )(Z, Wp.T, bp, Al, Ar)


_MESH = plsc.VectorSubcoreMesh(core_axis_name="c", subcore_axis_name="s")


@functools.partial(
    pl.kernel,
    out_type=[
        jax.ShapeDtypeStruct((2 * N_NODES, HC), jnp.float32),
        jax.ShapeDtypeStruct((2 * N_NODES, HC), jnp.float32),
        jax.ShapeDtypeStruct((2 * NS * NCH * K, 16), jnp.float32),
    ],
    mesh=_MESH,
    compiler_params=pltpu.CompilerParams(
        needs_layout_passes=False, use_tc_tiling_on_sc=False),
    scratch_types=[
        [pltpu.VMEM((K,), jnp.int32),
         pltpu.VMEM((K,), jnp.int32)],
        [pltpu.VMEM((K,), jnp.int32),
         pltpu.VMEM((K,), jnp.int32)],
        [pltpu.VMEM((K,), jnp.int32),
         pltpu.VMEM((K,), jnp.int32)],
        [pltpu.VMEM((K, 16), jnp.float32),
         pltpu.VMEM((K, 16), jnp.float32)],
        [pltpu.VMEM((K, 16), jnp.float32),
         pltpu.VMEM((K, 16), jnp.float32)],
        [pltpu.VMEM((K, 16), jnp.float32),
         pltpu.VMEM((K, 16), jnp.float32)],
        [pltpu.SemaphoreType.DMA,
         pltpu.SemaphoreType.DMA],
        [pltpu.SemaphoreType.DMA,
         pltpu.SemaphoreType.DMA],
        pltpu.VMEM((SUB, 16), jnp.float32),
        [pltpu.VMEM((K2,), jnp.int32),
         pltpu.VMEM((K2,), jnp.int32)],
        [pltpu.VMEM((K2,), jnp.int32),
         pltpu.VMEM((K2,), jnp.int32)],
        [pltpu.VMEM((K2, 16), jnp.float32),
         pltpu.VMEM((K2, 16), jnp.float32)],
        [pltpu.VMEM((K2, HC), jnp.float32),
         pltpu.VMEM((K2, HC), jnp.float32)],
        [pltpu.SemaphoreType.DMA,
         pltpu.SemaphoreType.DMA],
        [pltpu.SemaphoreType.DMA,
         pltpu.SemaphoreType.DMA],
        pltpu.VMEM_SHARED((N_NODES, 16), jnp.float32),
        pltpu.VMEM_SHARED((N_NODES, HC), jnp.float32),
    ],
)
def _sc_edges(idx_cat, tl2, trm2, zp2, rout, qbuf, tbuf,
              i_sq, i_dst, i_dq, tl_b, trm_b, tq, semg, semh, s_b,
              i_s2, i_dq2, tq2, rows2, semq, semt,
              S_sh, R_sh):
    cid = lax.axis_index("c")
    sid = lax.axis_index("s")
    iota = lax.iota(jnp.int32, 16)
    zeros16 = jnp.zeros((16,), jnp.float32)

    nbase = sid * RPT
    nsub = jnp.where(sid < NS - 1, RPT // SUB,
                     (N_NODES - (NS - 1) * RPT) // SUB)

    tile_s = sid * EPT
    tile_d = N_EDGES + sid * EPT
    tile_t = (cid * NS + sid) * EPT

    rows0 = rows2[0]

    def _zero_rows(i, _):
        for j in range(HC // 16):
            rows0[i, pl.ds(j * 16, 16)] = zeros16
        s_b[i, :] = zeros16
        return 0
    lax.fori_loop(0, SUB, _zero_rows, 0)

    def _zero_stripe(s, _):
        off = nbase + s * SUB
        pltpu.sync_copy(rows0.at[pl.ds(0, SUB)], R_sh.at[pl.ds(off, SUB)])
        pltpu.sync_copy(s_b, S_sh.at[pl.ds(off, SUB)])
        return 0
    lax.fori_loop(0, nsub, _zero_stripe, 0)
    plsc.subcore_barrier()

    def _p1_stage(ch, sl):
        pltpu.sync_copy(idx_cat.at[pl.ds(tile_s + ch * K, K)], i_sq[sl])
        pltpu.sync_copy(idx_cat.at[pl.ds(tile_d + ch * K, K)], i_dst[sl])
        for g in range(G):
            sv = i_sq[sl][pl.ds(g * 16, 16)]
            dv = i_dst[sl][pl.ds(g * 16, 16)]
            i_sq[sl][pl.ds(g * 16, 16)] = sv + cid * N_NODES
            i_dq[sl][pl.ds(g * 16, 16)] = dv + cid * N_NODES
        pltpu.async_copy(tl2.at[i_sq[sl]], tl_b[sl], semg[sl])
        pltpu.async_copy(trm2.at[i_dq[sl]], trm_b[sl], semh[sl])

    def _p1_finish(ch, sl):
        pltpu.make_async_copy(tl2.at[pl.ds(0, K)], tl_b[sl], semg[sl]).wait()
        pltpu.make_async_copy(trm2.at[pl.ds(0, K)], trm_b[sl], semh[sl]).wait()
        for g in range(G):
            ri = iota + g * 16
            for h in range(HH):
                hc = jnp.full((16,), h, jnp.int32)
                el = plsc.load_gather(tl_b[sl], [ri, hc])
                er = plsc.load_gather(trm_b[sl], [ri, hc])
                m = plsc.load_gather(
                    trm_b[sl], [ri, jnp.full((16,), HH + h, jnp.int32)])
                t = jnp.exp(_leaky(el + er) - m)
                for j in range(4):
                    plsc.store_scatter(
                        tq[sl], [ri, jnp.full((16,), j * HH + h, jnp.int32)], t)
        pltpu.sync_copy(tq[sl], S_sh.at[i_dst[sl]], add=True)
        pltpu.sync_copy(tq[sl], tbuf.at[pl.ds(tile_t + ch * K, K)])

    _p1_stage(0, 0)

    def _p1_pair(p, _):
        _p1_stage(2 * p + 1, 1)
        _p1_finish(2 * p, 0)

        @pl.when(p < NCH // 2 - 1)
        def _():
            _p1_stage(2 * p + 2, 0)
        _p1_finish(2 * p + 1, 1)
        return 0
    lax.fori_loop(0, NCH // 2, _p1_pair, 0)
    plsc.subcore_barrier()

    def _q_sub(s, _):
        off = nbase + s * SUB
        pltpu.sync_copy(zp2.at[pl.ds(cid * N_NODES + off, SUB)],
                        rows0.at[pl.ds(0, SUB)])
        pltpu.sync_copy(S_sh.at[pl.ds(off, SUB)], s_b)
        for i in range(SUB):
            sq = 1.0 / (s_b[i, :] + 1e-16)
            for s16 in range(HC // 16):
                v = rows0[i, pl.ds(s16 * 16, 16)]
                rows0[i, pl.ds(s16 * 16, 16)] = v * sq
        pltpu.sync_copy(rows0.at[pl.ds(0, SUB)],
                        qbuf.at[pl.ds(cid * N_NODES + off, SUB)])
        return 0
    lax.fori_loop(0, nsub, _q_sub, 0)
    plsc.subcore_barrier()

    def _p2_issue(ch, sl):
        pltpu.sync_copy(idx_cat.at[pl.ds(tile_s + ch * K2, K2)], i_s2[sl])
        pltpu.sync_copy(idx_cat.at[pl.ds(tile_d + ch * K2, K2)], i_dq2[sl])
        for g in range(G2):
            dv = i_dq2[sl][pl.ds(g * 16, 16)]
            i_dq2[sl][pl.ds(g * 16, 16)] = dv + cid * N_NODES
        pltpu.async_copy(tbuf.at[pl.ds(tile_t + ch * K2, K2)], tq2[sl], semt[sl])
        pltpu.async_copy(qbuf.at[i_dq2[sl]], rows2[sl], semq[sl])

    def _p2_finish(ch, sl):
        pltpu.make_async_copy(tbuf.at[pl.ds(tile_t + ch * K2, K2)], tq2[sl],
                              semt[sl]).wait()
        pltpu.make_async_copy(qbuf.at[pl.ds(0, K2)], rows2[sl], semq[sl]).wait()

        def _scale_grp(g, _):
            for j in range(16):
                i = g * 16 + j
                tv = tq2[sl][i, :]
                for s16 in range(HC // 16):
                    v = rows2[sl][i, pl.ds(s16 * 16, 16)]
                    rows2[sl][i, pl.ds(s16 * 16, 16)] = v * tv
            return 0
        lax.fori_loop(0, G2, _scale_grp, 0)
        pltpu.sync_copy(rows2[sl], R_sh.at[i_s2[sl]], add=True)

    _p2_issue(0, 0)

    def _p2_pair(p, _):
        _p2_issue(2 * p + 1, 1)
        _p2_finish(2 * p, 0)
        _p2_issue(2 * p + 2, 0)
        _p2_finish(2 * p + 1, 1)
        return 0
    lax.fori_loop(0, NCH2 // 2, _p2_pair, 0)
    _p2_finish(NCH2 - 1, 0)
    plsc.subcore_barrier()

    def _r_out(s, _):
        off = nbase + s * SUB
        pltpu.sync_copy(R_sh.at[pl.ds(off, SUB)], rows0.at[pl.ds(0, SUB)])
        pltpu.sync_copy(rows0.at[pl.ds(0, SUB)],
                        rout.at[pl.ds(cid * N_NODES + off, SUB)])
        return 0
    lax.fori_loop(0, nsub, _r_out, 0)


def kernel(index, n, Z, W, b, a_l, a_r):
    num_nodes = Z.shape[0]
    zp2, tl2, trm2 = _dense_prep(Z, W, b, a_l, a_r)
    idx_cat = index.astype(jnp.int32).reshape(-1)
    rout, _, _ = _sc_edges(idx_cat, tl2, trm2, zp2)
    rst = jnp.concatenate(
        [rout[:num_nodes].reshape(num_nodes, OUT_SIZE, HH),
         rout[num_nodes:].reshape(num_nodes, OUT_SIZE, HH)], axis=2)
    return rst * (jnp.asarray(n, dtype=rst.dtype) / num_nodes)


# speedup vs baseline: 1.1035x; 1.0184x over previous
import functools

import jax
import jax.numpy as jnp
from jax import lax
from jax.experimental import pallas as pl
from jax.experimental.pallas import tpu as pltpu
from jax.experimental.pallas import tpu_sc as plsc

N_NODES = 10000
N_EDGES = 320000
IN_SIZE = 128
OUT_SIZE = 16
NUM_HEADS = 8
HH = NUM_HEADS // 2
HC = OUT_SIZE * HH

NS = 16
EPT = N_EDGES // NS
K = 400
NCH = EPT // K
G = K // 16
K2 = 160
NCH2 = EPT // K2
G2 = K2 // 16
RPT = 640
SUB = 80


def _leaky(x):
    return jnp.maximum(x, 0.01 * x)


def _prep_body(z_ref, wt_ref, b_ref, al_ref, ar_ref,
               zp2_ref, tl2_ref, trm2_ref):
    n = z_ref.shape[0]
    z = z_ref[...]
    zp = jnp.dot(z, wt_ref[...], preferred_element_type=jnp.float32)
    zp = zp + b_ref[...][None, :]
    el = jnp.dot(zp, al_ref[...], preferred_element_type=jnp.float32)
    er = jnp.dot(zp, ar_ref[...], preferred_element_type=jnp.float32)
    gmax = jnp.max(el, axis=0, keepdims=True)
    m = _leaky(gmax + er)
    zeros12 = jnp.zeros((n, 16 - HH), jnp.float32)
    zp2_ref[0:n] = zp[:, :HC]
    zp2_ref[n:] = zp[:, HC:]
    tl2_ref[0:n] = jnp.concatenate([el[:, :HH], zeros12], axis=1)
    tl2_ref[n:] = jnp.concatenate([el[:, HH:], zeros12], axis=1)
    zeros8 = jnp.zeros((n, 2 * HH), jnp.float32)
    trm2_ref[0:n] = jnp.concatenate([er[:, :HH], m[:, :HH], zeros8], axis=1)
    trm2_ref[n:] = jnp.concatenate([er[:, HH:], m[:, HH:], zeros8], axis=1)


def _dense_prep(Z, W, b, a_l, a_r):
    n = Z.shape[0]
    nf = OUT_SIZE * NUM_HEADS
    jp = jnp.arange(nf)
    c = jp // HC
    f = (jp % HC) // HH
    hp = jp % HH
    perm = f * NUM_HEADS + c * HH + hp
    Wp = W[perm]
    bp = b[perm]
    h_of = c * HH + hp
    Al = jnp.zeros((nf, NUM_HEADS), jnp.float32).at[jp, h_of].set(a_l[0][f, h_of])
    Ar = jnp.zeros((nf, NUM_HEADS), jnp.float32).at[jp, h_of].set(a_r[0][f, h_of])
    return pl.pallas_call(
        _prep_body,
        out_shape=[
            jax.ShapeDtypeStruct((2 * n, HC), jnp.float32),
            jax.ShapeDtypeStruct((2 * n, 16), jnp.float32),
            jax.ShapeDtypeStruct((2 * n, 16), jnp.float32),
        ],
    )(Z, Wp.T, bp, Al, Ar)


_MESH = plsc.VectorSubcoreMesh(core_axis_name="c", subcore_axis_name="s")


@functools.partial(
    pl.kernel,
    out_type=[
        jax.ShapeDtypeStruct((2 * N_NODES, HC), jnp.float32),
        jax.ShapeDtypeStruct((2 * N_NODES, HC), jnp.float32),
        jax.ShapeDtypeStruct((2 * NS * NCH * K, 16), jnp.float32),
    ],
    mesh=_MESH,
    compiler_params=pltpu.CompilerParams(
        needs_layout_passes=False, use_tc_tiling_on_sc=False),
    scratch_types=[
        [pltpu.VMEM((K,), jnp.int32),
         pltpu.VMEM((K,), jnp.int32)],
        [pltpu.VMEM((K,), jnp.int32),
         pltpu.VMEM((K,), jnp.int32)],
        [pltpu.VMEM((K,), jnp.int32),
         pltpu.VMEM((K,), jnp.int32)],
        [pltpu.VMEM((K, 16), jnp.float32),
         pltpu.VMEM((K, 16), jnp.float32)],
        [pltpu.VMEM((K, 16), jnp.float32),
         pltpu.VMEM((K, 16), jnp.float32)],
        [pltpu.VMEM((K, 16), jnp.float32),
         pltpu.VMEM((K, 16), jnp.float32)],
        [pltpu.SemaphoreType.DMA,
         pltpu.SemaphoreType.DMA],
        [pltpu.SemaphoreType.DMA,
         pltpu.SemaphoreType.DMA],
        pltpu.VMEM((SUB, 16), jnp.float32),
        [pltpu.VMEM((K2,), jnp.int32),
         pltpu.VMEM((K2,), jnp.int32)],
        [pltpu.VMEM((K2,), jnp.int32),
         pltpu.VMEM((K2,), jnp.int32)],
        [pltpu.VMEM((K2, 16), jnp.float32),
         pltpu.VMEM((K2, 16), jnp.float32)],
        [pltpu.VMEM((K2, HC), jnp.float32),
         pltpu.VMEM((K2, HC), jnp.float32)],
        [pltpu.SemaphoreType.DMA,
         pltpu.SemaphoreType.DMA],
        [pltpu.SemaphoreType.DMA,
         pltpu.SemaphoreType.DMA],
        [pltpu.SemaphoreType.DMA,
         pltpu.SemaphoreType.DMA],
        [pltpu.SemaphoreType.DMA,
         pltpu.SemaphoreType.DMA],
        [pltpu.SemaphoreType.DMA,
         pltpu.SemaphoreType.DMA],
        pltpu.VMEM_SHARED((N_NODES, 16), jnp.float32),
        pltpu.VMEM_SHARED((N_NODES, HC), jnp.float32),
    ],
)
def _sc_edges(idx_cat, tl2, trm2, zp2, rout, qbuf, tbuf,
              i_sq, i_dst, i_dq, tl_b, trm_b, tq, semg, semh, s_b,
              i_s2, i_dq2, tq2, rows2, semq, semt, semS, semW, semR,
              S_sh, R_sh):
    cid = lax.axis_index("c")
    sid = lax.axis_index("s")
    iota = lax.iota(jnp.int32, 16)
    zeros16 = jnp.zeros((16,), jnp.float32)

    nbase = sid * RPT
    nsub = jnp.where(sid < NS - 1, RPT // SUB,
                     (N_NODES - (NS - 1) * RPT) // SUB)

    tile_s = sid * EPT
    tile_d = N_EDGES + sid * EPT
    tile_t = (cid * NS + sid) * EPT

    rows0 = rows2[0]

    def _zero_rows(i, _):
        for j in range(HC // 16):
            rows0[i, pl.ds(j * 16, 16)] = zeros16
        s_b[i, :] = zeros16
        return 0
    lax.fori_loop(0, SUB, _zero_rows, 0)

    def _zero_stripe(s, _):
        off = nbase + s * SUB
        pltpu.sync_copy(rows0.at[pl.ds(0, SUB)], R_sh.at[pl.ds(off, SUB)])
        pltpu.sync_copy(s_b, S_sh.at[pl.ds(off, SUB)])
        return 0
    lax.fori_loop(0, nsub, _zero_stripe, 0)
    plsc.subcore_barrier()

    def _p1_stage(ch, sl):
        @pl.when(ch >= 2)
        def _():
            pltpu.make_async_copy(tq[sl], S_sh.at[pl.ds(0, K)], semS[sl]).wait()
            pltpu.make_async_copy(tq[sl], tbuf.at[pl.ds(0, K)], semW[sl]).wait()
        pltpu.sync_copy(idx_cat.at[pl.ds(tile_s + ch * K, K)], i_sq[sl])
        pltpu.sync_copy(idx_cat.at[pl.ds(tile_d + ch * K, K)], i_dst[sl])
        for g in range(G):
            sv = i_sq[sl][pl.ds(g * 16, 16)]
            dv = i_dst[sl][pl.ds(g * 16, 16)]
            i_sq[sl][pl.ds(g * 16, 16)] = sv + cid * N_NODES
            i_dq[sl][pl.ds(g * 16, 16)] = dv + cid * N_NODES
        pltpu.async_copy(tl2.at[i_sq[sl]], tl_b[sl], semg[sl])
        pltpu.async_copy(trm2.at[i_dq[sl]], trm_b[sl], semh[sl])

    def _p1_finish(ch, sl):
        pltpu.make_async_copy(tl2.at[pl.ds(0, K)], tl_b[sl], semg[sl]).wait()
        pltpu.make_async_copy(trm2.at[pl.ds(0, K)], trm_b[sl], semh[sl]).wait()
        for g in range(G):
            ri = iota + g * 16
            for h in range(HH):
                hc = jnp.full((16,), h, jnp.int32)
                el = plsc.load_gather(tl_b[sl], [ri, hc])
                er = plsc.load_gather(trm_b[sl], [ri, hc])
                m = plsc.load_gather(
                    trm_b[sl], [ri, jnp.full((16,), HH + h, jnp.int32)])
                t = jnp.exp(_leaky(el + er) - m)
                for j in range(4):
                    plsc.store_scatter(
                        tq[sl], [ri, jnp.full((16,), j * HH + h, jnp.int32)], t)
        pltpu.async_copy(tq[sl], S_sh.at[i_dst[sl]], semS[sl], add=True)
        pltpu.async_copy(tq[sl], tbuf.at[pl.ds(tile_t + ch * K, K)], semW[sl])

    _p1_stage(0, 0)

    def _p1_pair(p, _):
        _p1_stage(2 * p + 1, 1)
        _p1_finish(2 * p, 0)

        @pl.when(p < NCH // 2 - 1)
        def _():
            _p1_stage(2 * p + 2, 0)
        _p1_finish(2 * p + 1, 1)
        return 0
    lax.fori_loop(0, NCH // 2, _p1_pair, 0)
    for sl in (0, 1):
        pltpu.make_async_copy(tq[sl], S_sh.at[pl.ds(0, K)], semS[sl]).wait()
        pltpu.make_async_copy(tq[sl], tbuf.at[pl.ds(0, K)], semW[sl]).wait()
    plsc.subcore_barrier()

    def _q_sub(s, _):
        off = nbase + s * SUB
        pltpu.sync_copy(zp2.at[pl.ds(cid * N_NODES + off, SUB)],
                        rows0.at[pl.ds(0, SUB)])
        pltpu.sync_copy(S_sh.at[pl.ds(off, SUB)], s_b)
        for i in range(SUB):
            sq = 1.0 / (s_b[i, :] + 1e-16)
            for s16 in range(HC // 16):
                v = rows0[i, pl.ds(s16 * 16, 16)]
                rows0[i, pl.ds(s16 * 16, 16)] = v * sq
        pltpu.sync_copy(rows0.at[pl.ds(0, SUB)],
                        qbuf.at[pl.ds(cid * N_NODES + off, SUB)])
        return 0
    lax.fori_loop(0, nsub, _q_sub, 0)
    plsc.subcore_barrier()

    def _p2_issue(ch, sl):
        @pl.when(ch >= 2)
        def _():
            pltpu.make_async_copy(rows2[sl], R_sh.at[pl.ds(0, K2)],
                                  semR[sl]).wait()
        pltpu.sync_copy(idx_cat.at[pl.ds(tile_s + ch * K2, K2)], i_s2[sl])
        pltpu.sync_copy(idx_cat.at[pl.ds(tile_d + ch * K2, K2)], i_dq2[sl])
        for g in range(G2):
            dv = i_dq2[sl][pl.ds(g * 16, 16)]
            i_dq2[sl][pl.ds(g * 16, 16)] = dv + cid * N_NODES
        pltpu.async_copy(tbuf.at[pl.ds(tile_t + ch * K2, K2)], tq2[sl], semt[sl])
        pltpu.async_copy(qbuf.at[i_dq2[sl]], rows2[sl], semq[sl])

    def _p2_finish(ch, sl):
        pltpu.make_async_copy(tbuf.at[pl.ds(tile_t + ch * K2, K2)], tq2[sl],
                              semt[sl]).wait()
        pltpu.make_async_copy(qbuf.at[pl.ds(0, K2)], rows2[sl], semq[sl]).wait()

        def _scale_grp(g, _):
            for j in range(16):
                i = g * 16 + j
                tv = tq2[sl][i, :]
                for s16 in range(HC // 16):
                    v = rows2[sl][i, pl.ds(s16 * 16, 16)]
                    rows2[sl][i, pl.ds(s16 * 16, 16)] = v * tv
            return 0
        lax.fori_loop(0, G2, _scale_grp, 0)
        pltpu.async_copy(rows2[sl], R_sh.at[i_s2[sl]], semR[sl], add=True)

    _p2_issue(0, 0)

    def _p2_pair(p, _):
        _p2_issue(2 * p + 1, 1)
        _p2_finish(2 * p, 0)
        _p2_issue(2 * p + 2, 0)
        _p2_finish(2 * p + 1, 1)
        return 0
    lax.fori_loop(0, NCH2 // 2, _p2_pair, 0)
    _p2_finish(NCH2 - 1, 0)
    for sl in (0, 1):
        pltpu.make_async_copy(rows2[sl], R_sh.at[pl.ds(0, K2)],
                              semR[sl]).wait()
    plsc.subcore_barrier()

    def _r_out(s, _):
        off = nbase + s * SUB
        pltpu.sync_copy(R_sh.at[pl.ds(off, SUB)], rows0.at[pl.ds(0, SUB)])
        pltpu.sync_copy(rows0.at[pl.ds(0, SUB)],
                        rout.at[pl.ds(cid * N_NODES + off, SUB)])
        return 0
    lax.fori_loop(0, nsub, _r_out, 0)


def kernel(index, n, Z, W, b, a_l, a_r):
    num_nodes = Z.shape[0]
    zp2, tl2, trm2 = _dense_prep(Z, W, b, a_l, a_r)
    idx_cat = index.astype(jnp.int32).reshape(-1)
    rout, _, _ = _sc_edges(idx_cat, tl2, trm2, zp2)
    rst = jnp.concatenate(
        [rout[:num_nodes].reshape(num_nodes, OUT_SIZE, HH),
         rout[num_nodes:].reshape(num_nodes, OUT_SIZE, HH)], axis=2)
    return rst * (jnp.asarray(n, dtype=rst.dtype) / num_nodes)


# speedup vs baseline: 1.2461x; 1.1292x over previous
import functools

import jax
import jax.numpy as jnp
from jax import lax
from jax.experimental import pallas as pl
from jax.experimental.pallas import tpu as pltpu
from jax.experimental.pallas import tpu_sc as plsc

N_NODES = 10000
N_EDGES = 320000
IN_SIZE = 128
OUT_SIZE = 16
NUM_HEADS = 8
HH = NUM_HEADS // 2
HC = OUT_SIZE * HH

NS = 16
EPT = N_EDGES // NS
K = 400
NCH = EPT // K
G = K // 16
K2 = 160
NCH2 = EPT // K2
G2 = K2 // 16
RPT = 640
SUB = 80


def _leaky(x):
    return jnp.maximum(x, 0.01 * x)


def _prep_body(z_ref, wt_ref, b_ref, al_ref, ar_ref,
               zp2_ref, tl2_ref, trm2_ref):
    n = z_ref.shape[0]
    z = z_ref[...]
    zp = jnp.dot(z, wt_ref[...], preferred_element_type=jnp.float32)
    zp = zp + b_ref[...][None, :]
    el = jnp.dot(zp, al_ref[...], preferred_element_type=jnp.float32)
    er = jnp.dot(zp, ar_ref[...], preferred_element_type=jnp.float32)
    gmax = jnp.max(el, axis=0, keepdims=True)
    m = _leaky(gmax + er)
    zeros12 = jnp.zeros((n, 16 - HH), jnp.float32)
    zp2_ref[0:n] = zp[:, :HC]
    zp2_ref[n:] = zp[:, HC:]
    tl2_ref[0:n] = jnp.concatenate([el[:, :HH], zeros12], axis=1)
    tl2_ref[n:] = jnp.concatenate([el[:, HH:], zeros12], axis=1)
    zeros8 = jnp.zeros((n, 2 * HH), jnp.float32)
    trm2_ref[0:n] = jnp.concatenate([er[:, :HH], m[:, :HH], zeros8], axis=1)
    trm2_ref[n:] = jnp.concatenate([er[:, HH:], m[:, HH:], zeros8], axis=1)


def _dense_prep(Z, W, b, a_l, a_r):
    n = Z.shape[0]
    nf = OUT_SIZE * NUM_HEADS
    jp = jnp.arange(nf)
    c = jp // HC
    f = (jp % HC) // HH
    hp = jp % HH
    perm = f * NUM_HEADS + c * HH + hp
    Wp = W[perm]
    bp = b[perm]
    h_of = c * HH + hp
    Al = jnp.zeros((nf, NUM_HEADS), jnp.float32).at[jp, h_of].set(a_l[0][f, h_of])
    Ar = jnp.zeros((nf, NUM_HEADS), jnp.float32).at[jp, h_of].set(a_r[0][f, h_of])
    return pl.pallas_call(
        _prep_body,
        out_shape=[
            jax.ShapeDtypeStruct((2 * n, HC), jnp.float32),
            jax.ShapeDtypeStruct((2 * n, 16), jnp.float32),
            jax.ShapeDtypeStruct((2 * n, 16), jnp.float32),
        ],
    )(Z, Wp.T, bp, Al, Ar)


_MESH = plsc.VectorSubcoreMesh(core_axis_name="c", subcore_axis_name="s")


@functools.partial(
    pl.kernel,
    out_type=[
        jax.ShapeDtypeStruct((2 * N_NODES, HC), jnp.float32),
        jax.ShapeDtypeStruct((2 * N_NODES, HC), jnp.float32),
        jax.ShapeDtypeStruct((2 * NS * NCH * K, 16), jnp.float32),
    ],
    mesh=_MESH,
    compiler_params=pltpu.CompilerParams(
        needs_layout_passes=False, use_tc_tiling_on_sc=False),
    scratch_types=[
        [pltpu.VMEM((K,), jnp.int32),
         pltpu.VMEM((K,), jnp.int32)],
        [pltpu.VMEM((K,), jnp.int32),
         pltpu.VMEM((K,), jnp.int32)],
        [pltpu.VMEM((K,), jnp.int32),
         pltpu.VMEM((K,), jnp.int32)],
        [pltpu.VMEM((K, 16), jnp.float32),
         pltpu.VMEM((K, 16), jnp.float32)],
        [pltpu.VMEM((K, 16), jnp.float32),
         pltpu.VMEM((K, 16), jnp.float32)],
        [pltpu.VMEM((K, 16), jnp.float32),
         pltpu.VMEM((K, 16), jnp.float32)],
        [pltpu.SemaphoreType.DMA,
         pltpu.SemaphoreType.DMA],
        [pltpu.SemaphoreType.DMA,
         pltpu.SemaphoreType.DMA],
        pltpu.VMEM((SUB, 16), jnp.float32),
        [pltpu.VMEM((K2,), jnp.int32),
         pltpu.VMEM((K2,), jnp.int32)],
        [pltpu.VMEM((K2,), jnp.int32),
         pltpu.VMEM((K2,), jnp.int32)],
        [pltpu.VMEM((K2, 16), jnp.float32),
         pltpu.VMEM((K2, 16), jnp.float32)],
        [pltpu.VMEM((K2, HC), jnp.float32),
         pltpu.VMEM((K2, HC), jnp.float32)],
        [pltpu.SemaphoreType.DMA,
         pltpu.SemaphoreType.DMA],
        [pltpu.SemaphoreType.DMA,
         pltpu.SemaphoreType.DMA],
        [pltpu.SemaphoreType.DMA,
         pltpu.SemaphoreType.DMA],
        [pltpu.SemaphoreType.DMA,
         pltpu.SemaphoreType.DMA],
        [pltpu.SemaphoreType.DMA,
         pltpu.SemaphoreType.DMA],
        pltpu.VMEM((2 * K2,), jnp.int32),
        pltpu.VMEM((2 * K2,), jnp.int32),
        pltpu.VMEM_SHARED((N_NODES, 16), jnp.float32),
        pltpu.VMEM_SHARED((N_NODES, HC), jnp.float32),
    ],
)
def _sc_edges(idx_cat, tl2, trm2, zp2, rout, qbuf, tbuf,
              i_sq, i_dst, i_dq, tl_b, trm_b, tq, semg, semh, s_b,
              i_s2, i_dq2, tq2, rows2, semq, semt, semS, semW, semR,
              pairS, pairD, S_sh, R_sh):
    cid = lax.axis_index("c")
    sid = lax.axis_index("s")
    iota = lax.iota(jnp.int32, 16)
    zeros16 = jnp.zeros((16,), jnp.float32)

    nbase = sid * RPT
    nsub = jnp.where(sid < NS - 1, RPT // SUB,
                     (N_NODES - (NS - 1) * RPT) // SUB)

    tile_s = sid * EPT
    tile_d = N_EDGES + sid * EPT
    tile_t = (cid * NS + sid) * EPT

    rows0 = rows2[0]

    def _zero_rows(i, _):
        for j in range(HC // 16):
            rows0[i, pl.ds(j * 16, 16)] = zeros16
        s_b[i, :] = zeros16
        return 0
    lax.fori_loop(0, SUB, _zero_rows, 0)

    def _zero_stripe(s, _):
        off = nbase + s * SUB
        pltpu.sync_copy(rows0.at[pl.ds(0, SUB)], R_sh.at[pl.ds(off, SUB)])
        pltpu.sync_copy(s_b, S_sh.at[pl.ds(off, SUB)])
        return 0
    lax.fori_loop(0, nsub, _zero_stripe, 0)
    plsc.subcore_barrier()

    def _p1_stage(ch, sl):
        @pl.when(ch >= 2)
        def _():
            pltpu.make_async_copy(tq[sl], S_sh.at[pl.ds(0, K)], semS[sl]).wait()
            pltpu.make_async_copy(tq[sl], tbuf.at[pl.ds(0, K)], semW[sl]).wait()
        pltpu.sync_copy(idx_cat.at[pl.ds(tile_s + ch * K, K)], i_sq[sl])
        pltpu.sync_copy(idx_cat.at[pl.ds(tile_d + ch * K, K)], i_dst[sl])
        for g in range(G):
            sv = i_sq[sl][pl.ds(g * 16, 16)]
            dv = i_dst[sl][pl.ds(g * 16, 16)]
            i_sq[sl][pl.ds(g * 16, 16)] = sv + cid * N_NODES
            i_dq[sl][pl.ds(g * 16, 16)] = dv + cid * N_NODES
        pltpu.async_copy(tl2.at[i_sq[sl]], tl_b[sl], semg[sl])
        pltpu.async_copy(trm2.at[i_dq[sl]], trm_b[sl], semh[sl])

    def _p1_finish(ch, sl):
        pltpu.make_async_copy(tl2.at[pl.ds(0, K)], tl_b[sl], semg[sl]).wait()
        pltpu.make_async_copy(trm2.at[pl.ds(0, K)], trm_b[sl], semh[sl]).wait()
        for g in range(G):
            ri = iota + g * 16
            for h in range(HH):
                hc = jnp.full((16,), h, jnp.int32)
                el = plsc.load_gather(tl_b[sl], [ri, hc])
                er = plsc.load_gather(trm_b[sl], [ri, hc])
                m = plsc.load_gather(
                    trm_b[sl], [ri, jnp.full((16,), HH + h, jnp.int32)])
                t = jnp.exp(_leaky(el + er) - m)
                for j in range(4):
                    plsc.store_scatter(
                        tq[sl], [ri, jnp.full((16,), j * HH + h, jnp.int32)], t)
        pltpu.async_copy(tq[sl], S_sh.at[i_dst[sl]], semS[sl], add=True)
        pltpu.async_copy(tq[sl], tbuf.at[pl.ds(tile_t + ch * K, K)], semW[sl])

    _p1_stage(0, 0)

    def _p1_pair(p, _):
        _p1_stage(2 * p + 1, 1)
        _p1_finish(2 * p, 0)

        @pl.when(p < NCH // 2 - 1)
        def _():
            _p1_stage(2 * p + 2, 0)
        _p1_finish(2 * p + 1, 1)
        return 0
    lax.fori_loop(0, NCH // 2, _p1_pair, 0)
    for sl in (0, 1):
        pltpu.make_async_copy(tq[sl], S_sh.at[pl.ds(0, K)], semS[sl]).wait()
        pltpu.make_async_copy(tq[sl], tbuf.at[pl.ds(0, K)], semW[sl]).wait()
    plsc.subcore_barrier()

    def _q_sub(s, _):
        off = nbase + s * SUB
        pltpu.sync_copy(zp2.at[pl.ds(cid * N_NODES + off, SUB)],
                        rows0.at[pl.ds(0, SUB)])
        pltpu.sync_copy(S_sh.at[pl.ds(off, SUB)], s_b)
        for i in range(SUB):
            sq = 1.0 / (s_b[i, :] + 1e-16)
            for s16 in range(HC // 16):
                v = rows0[i, pl.ds(s16 * 16, 16)]
                rows0[i, pl.ds(s16 * 16, 16)] = v * sq
        pltpu.sync_copy(rows0.at[pl.ds(0, SUB)],
                        qbuf.at[pl.ds(cid * N_NODES + off, SUB)])
        return 0
    lax.fori_loop(0, nsub, _q_sub, 0)
    plsc.subcore_barrier()

    def _p2_issue(ch, sl, poff=None):
        @pl.when(ch >= 2)
        def _():
            pltpu.make_async_copy(rows2[sl], R_sh.at[pl.ds(0, K2)],
                                  semR[sl]).wait()
        if poff is None:
            pltpu.sync_copy(idx_cat.at[pl.ds(tile_s + ch * K2, K2)], i_s2[sl])
            pltpu.sync_copy(idx_cat.at[pl.ds(tile_d + ch * K2, K2)], i_dq2[sl])
            for g in range(G2):
                dv = i_dq2[sl][pl.ds(g * 16, 16)]
                i_dq2[sl][pl.ds(g * 16, 16)] = dv + cid * N_NODES
        else:
            for g in range(G2):
                i_s2[sl][pl.ds(g * 16, 16)] = pairS[pl.ds(poff + g * 16, 16)]
                dv = pairD[pl.ds(poff + g * 16, 16)]
                i_dq2[sl][pl.ds(g * 16, 16)] = dv + cid * N_NODES
        pltpu.async_copy(tbuf.at[pl.ds(tile_t + ch * K2, K2)], tq2[sl], semt[sl])
        pltpu.async_copy(qbuf.at[i_dq2[sl]], rows2[sl], semq[sl])

    def _p2_finish(ch, sl):
        pltpu.make_async_copy(tbuf.at[pl.ds(tile_t + ch * K2, K2)], tq2[sl],
                              semt[sl]).wait()
        pltpu.make_async_copy(qbuf.at[pl.ds(0, K2)], rows2[sl], semq[sl]).wait()

        def _scale_grp(g, _):
            for j in range(16):
                i = g * 16 + j
                tv = tq2[sl][i, :]
                for s16 in range(HC // 16):
                    v = rows2[sl][i, pl.ds(s16 * 16, 16)]
                    rows2[sl][i, pl.ds(s16 * 16, 16)] = v * tv
            return 0
        lax.fori_loop(0, G2, _scale_grp, 0)
        pltpu.async_copy(rows2[sl], R_sh.at[i_s2[sl]], semR[sl], add=True)

    _p2_issue(0, 0)

    def _p2_pair(p, _):
        blk = (2 * p + 1) * K2
        pltpu.sync_copy(idx_cat.at[pl.ds(tile_s + blk, 2 * K2)], pairS)
        pltpu.sync_copy(idx_cat.at[pl.ds(tile_d + blk, 2 * K2)], pairD)
        _p2_issue(2 * p + 1, 1, 0)
        _p2_finish(2 * p, 0)
        _p2_issue(2 * p + 2, 0, K2)
        _p2_finish(2 * p + 1, 1)
        return 0
    lax.fori_loop(0, NCH2 // 2, _p2_pair, 0)
    _p2_finish(NCH2 - 1, 0)
    for sl in (0, 1):
        pltpu.make_async_copy(rows2[sl], R_sh.at[pl.ds(0, K2)],
                              semR[sl]).wait()
    plsc.subcore_barrier()

    def _r_out(s, _):
        off = nbase + s * SUB
        pltpu.sync_copy(R_sh.at[pl.ds(off, SUB)], rows0.at[pl.ds(0, SUB)])
        pltpu.sync_copy(rows0.at[pl.ds(0, SUB)],
                        rout.at[pl.ds(cid * N_NODES + off, SUB)])
        return 0
    lax.fori_loop(0, nsub, _r_out, 0)


def kernel(index, n, Z, W, b, a_l, a_r):
    num_nodes = Z.shape[0]
    zp2, tl2, trm2 = _dense_prep(Z, W, b, a_l, a_r)
    idx_cat = index.astype(jnp.int32).reshape(-1)
    rout, _, _ = _sc_edges(idx_cat, tl2, trm2, zp2)
    rst = jnp.concatenate(
        [rout[:num_nodes].reshape(num_nodes, OUT_SIZE, HH),
         rout[num_nodes:].reshape(num_nodes, OUT_SIZE, HH)], axis=2)
    return rst * (jnp.asarray(n, dtype=rst.dtype) / num_nodes)


# speedup vs baseline: 1.2836x; 1.0301x over previous
import functools

import jax
import jax.numpy as jnp
from jax import lax
from jax.experimental import pallas as pl
from jax.experimental.pallas import tpu as pltpu
from jax.experimental.pallas import tpu_sc as plsc

N_NODES = 10000
N_EDGES = 320000
IN_SIZE = 128
OUT_SIZE = 16
NUM_HEADS = 8
HH = NUM_HEADS // 2
HC = OUT_SIZE * HH

NS = 16
EPT = N_EDGES // NS
K = 400
NCH = EPT // K
G = K // 16
K2 = 160
NCH2 = EPT // K2
G2 = K2 // 16
RPT = 640
SUB = 80


def _leaky(x):
    return jnp.maximum(x, 0.01 * x)


def _prep_body(z_ref, wt_ref, b_ref, al_ref, ar_ref,
               zp2_ref, tl2_ref, trm2_ref):
    n = z_ref.shape[0]
    z = z_ref[...]
    zp = jnp.dot(z, wt_ref[...], preferred_element_type=jnp.float32)
    zp = zp + b_ref[...][None, :]
    el = jnp.dot(zp, al_ref[...], preferred_element_type=jnp.float32)
    er = jnp.dot(zp, ar_ref[...], preferred_element_type=jnp.float32)
    gmax = jnp.max(el, axis=0, keepdims=True)
    m = _leaky(gmax + er)
    zeros12 = jnp.zeros((n, 16 - HH), jnp.float32)
    zp2_ref[0:n] = zp[:, :HC]
    zp2_ref[n:] = zp[:, HC:]
    tl2_ref[0:n] = jnp.concatenate([el[:, :HH], zeros12], axis=1)
    tl2_ref[n:] = jnp.concatenate([el[:, HH:], zeros12], axis=1)
    zeros8 = jnp.zeros((n, 2 * HH), jnp.float32)
    trm2_ref[0:n] = jnp.concatenate([er[:, :HH], m[:, :HH], zeros8], axis=1)
    trm2_ref[n:] = jnp.concatenate([er[:, HH:], m[:, HH:], zeros8], axis=1)


def _dense_prep(Z, W, b, a_l, a_r):
    n = Z.shape[0]
    nf = OUT_SIZE * NUM_HEADS
    jp = jnp.arange(nf)
    c = jp // HC
    f = (jp % HC) // HH
    hp = jp % HH
    perm = f * NUM_HEADS + c * HH + hp
    Wp = W[perm]
    bp = b[perm]
    h_of = c * HH + hp
    Al = jnp.zeros((nf, NUM_HEADS), jnp.float32).at[jp, h_of].set(a_l[0][f, h_of])
    Ar = jnp.zeros((nf, NUM_HEADS), jnp.float32).at[jp, h_of].set(a_r[0][f, h_of])
    return pl.pallas_call(
        _prep_body,
        out_shape=[
            jax.ShapeDtypeStruct((2 * n, HC), jnp.float32),
            jax.ShapeDtypeStruct((2 * n, 16), jnp.float32),
            jax.ShapeDtypeStruct((2 * n, 16), jnp.float32),
        ],
    )(Z, Wp.T, bp, Al, Ar)


_MESH = plsc.VectorSubcoreMesh(core_axis_name="c", subcore_axis_name="s")


@functools.partial(
    pl.kernel,
    out_type=[
        jax.ShapeDtypeStruct((2 * N_NODES, HC), jnp.float32),
        jax.ShapeDtypeStruct((2 * N_NODES, HC), jnp.float32),
        jax.ShapeDtypeStruct((2 * NS * NCH * K, 16), jnp.float32),
    ],
    mesh=_MESH,
    compiler_params=pltpu.CompilerParams(
        needs_layout_passes=False, use_tc_tiling_on_sc=False),
    scratch_types=[
        [pltpu.VMEM((K,), jnp.int32),
         pltpu.VMEM((K,), jnp.int32)],
        [pltpu.VMEM((K,), jnp.int32),
         pltpu.VMEM((K,), jnp.int32)],
        [pltpu.VMEM((K,), jnp.int32),
         pltpu.VMEM((K,), jnp.int32)],
        [pltpu.VMEM((K, 16), jnp.float32),
         pltpu.VMEM((K, 16), jnp.float32)],
        [pltpu.VMEM((K, 16), jnp.float32),
         pltpu.VMEM((K, 16), jnp.float32)],
        [pltpu.VMEM((K, 16), jnp.float32),
         pltpu.VMEM((K, 16), jnp.float32)],
        [pltpu.SemaphoreType.DMA,
         pltpu.SemaphoreType.DMA],
        [pltpu.SemaphoreType.DMA,
         pltpu.SemaphoreType.DMA],
        pltpu.VMEM((SUB, 16), jnp.float32),
        [pltpu.VMEM((K2,), jnp.int32),
         pltpu.VMEM((K2,), jnp.int32)],
        [pltpu.VMEM((K2,), jnp.int32),
         pltpu.VMEM((K2,), jnp.int32)],
        [pltpu.VMEM((K2, 16), jnp.float32),
         pltpu.VMEM((K2, 16), jnp.float32)],
        [pltpu.VMEM((K2, HC), jnp.float32),
         pltpu.VMEM((K2, HC), jnp.float32)],
        [pltpu.SemaphoreType.DMA,
         pltpu.SemaphoreType.DMA],
        [pltpu.SemaphoreType.DMA,
         pltpu.SemaphoreType.DMA],
        [pltpu.SemaphoreType.DMA,
         pltpu.SemaphoreType.DMA],
        [pltpu.SemaphoreType.DMA,
         pltpu.SemaphoreType.DMA],
        [pltpu.SemaphoreType.DMA,
         pltpu.SemaphoreType.DMA],
        pltpu.VMEM((2 * K2,), jnp.int32),
        pltpu.VMEM((2 * K2,), jnp.int32),
        pltpu.VMEM((2 * K,), jnp.int32),
        pltpu.VMEM((2 * K,), jnp.int32),
        pltpu.VMEM_SHARED((N_NODES, 16), jnp.float32),
        pltpu.VMEM_SHARED((N_NODES, HC), jnp.float32),
    ],
)
def _sc_edges(idx_cat, tl2, trm2, zp2, rout, qbuf, tbuf,
              i_sq, i_dst, i_dq, tl_b, trm_b, tq, semg, semh, s_b,
              i_s2, i_dq2, tq2, rows2, semq, semt, semS, semW, semR,
              pairS, pairD, pairS1, pairD1, S_sh, R_sh):
    cid = lax.axis_index("c")
    sid = lax.axis_index("s")
    iota = lax.iota(jnp.int32, 16)
    zeros16 = jnp.zeros((16,), jnp.float32)

    nbase = sid * RPT
    nsub = jnp.where(sid < NS - 1, RPT // SUB,
                     (N_NODES - (NS - 1) * RPT) // SUB)

    tile_s = sid * EPT
    tile_d = N_EDGES + sid * EPT
    tile_t = (cid * NS + sid) * EPT

    rows0 = rows2[0]

    def _zero_rows(i, _):
        for j in range(HC // 16):
            rows0[i, pl.ds(j * 16, 16)] = zeros16
        s_b[i, :] = zeros16
        return 0
    lax.fori_loop(0, SUB, _zero_rows, 0)

    def _zero_stripe(s, _):
        off = nbase + s * SUB
        pltpu.sync_copy(rows0.at[pl.ds(0, SUB)], R_sh.at[pl.ds(off, SUB)])
        pltpu.sync_copy(s_b, S_sh.at[pl.ds(off, SUB)])
        return 0
    lax.fori_loop(0, nsub, _zero_stripe, 0)
    plsc.subcore_barrier()

    def _p1_stage(ch, sl, poff=None):
        @pl.when(ch >= 2)
        def _():
            pltpu.make_async_copy(tq[sl], S_sh.at[pl.ds(0, K)], semS[sl]).wait()
            pltpu.make_async_copy(tq[sl], tbuf.at[pl.ds(0, K)], semW[sl]).wait()
        if poff is None:
            pltpu.sync_copy(idx_cat.at[pl.ds(tile_s + ch * K, K)], i_sq[sl])
            pltpu.sync_copy(idx_cat.at[pl.ds(tile_d + ch * K, K)], i_dst[sl])
            for g in range(G):
                sv = i_sq[sl][pl.ds(g * 16, 16)]
                dv = i_dst[sl][pl.ds(g * 16, 16)]
                i_sq[sl][pl.ds(g * 16, 16)] = sv + cid * N_NODES
                i_dq[sl][pl.ds(g * 16, 16)] = dv + cid * N_NODES
        else:
            for g in range(G):
                sv = pairS1[pl.ds(poff + g * 16, 16)]
                dv = pairD1[pl.ds(poff + g * 16, 16)]
                i_sq[sl][pl.ds(g * 16, 16)] = sv + cid * N_NODES
                i_dst[sl][pl.ds(g * 16, 16)] = dv
                i_dq[sl][pl.ds(g * 16, 16)] = dv + cid * N_NODES
        pltpu.async_copy(tl2.at[i_sq[sl]], tl_b[sl], semg[sl])
        pltpu.async_copy(trm2.at[i_dq[sl]], trm_b[sl], semh[sl])

    def _p1_finish(ch, sl):
        pltpu.make_async_copy(tl2.at[pl.ds(0, K)], tl_b[sl], semg[sl]).wait()
        pltpu.make_async_copy(trm2.at[pl.ds(0, K)], trm_b[sl], semh[sl]).wait()
        for g in range(G):
            ri = iota + g * 16
            for h in range(HH):
                hc = jnp.full((16,), h, jnp.int32)
                el = plsc.load_gather(tl_b[sl], [ri, hc])
                er = plsc.load_gather(trm_b[sl], [ri, hc])
                m = plsc.load_gather(
                    trm_b[sl], [ri, jnp.full((16,), HH + h, jnp.int32)])
                t = jnp.exp(_leaky(el + er) - m)
                for j in range(4):
                    plsc.store_scatter(
                        tq[sl], [ri, jnp.full((16,), j * HH + h, jnp.int32)], t)
        pltpu.async_copy(tq[sl], S_sh.at[i_dst[sl]], semS[sl], add=True)
        pltpu.async_copy(tq[sl], tbuf.at[pl.ds(tile_t + ch * K, K)], semW[sl])

    _p1_stage(0, 0)

    def _p1_pair(p, _):
        blk = (2 * p + 1) * K
        pltpu.sync_copy(idx_cat.at[pl.ds(tile_s + blk, 2 * K)], pairS1)
        pltpu.sync_copy(idx_cat.at[pl.ds(tile_d + blk, 2 * K)], pairD1)
        _p1_stage(2 * p + 1, 1, 0)
        _p1_finish(2 * p, 0)

        @pl.when(p < NCH // 2 - 1)
        def _():
            _p1_stage(2 * p + 2, 0, K)
        _p1_finish(2 * p + 1, 1)
        return 0
    lax.fori_loop(0, NCH // 2, _p1_pair, 0)
    for sl in (0, 1):
        pltpu.make_async_copy(tq[sl], S_sh.at[pl.ds(0, K)], semS[sl]).wait()
        pltpu.make_async_copy(tq[sl], tbuf.at[pl.ds(0, K)], semW[sl]).wait()
    plsc.subcore_barrier()

    def _q_sub(s, _):
        off = nbase + s * SUB
        pltpu.sync_copy(zp2.at[pl.ds(cid * N_NODES + off, SUB)],
                        rows0.at[pl.ds(0, SUB)])
        pltpu.sync_copy(S_sh.at[pl.ds(off, SUB)], s_b)
        for i in range(SUB):
            sq = 1.0 / (s_b[i, :] + 1e-16)
            for s16 in range(HC // 16):
                v = rows0[i, pl.ds(s16 * 16, 16)]
                rows0[i, pl.ds(s16 * 16, 16)] = v * sq
        pltpu.sync_copy(rows0.at[pl.ds(0, SUB)],
                        qbuf.at[pl.ds(cid * N_NODES + off, SUB)])
        return 0
    lax.fori_loop(0, nsub, _q_sub, 0)
    plsc.subcore_barrier()

    def _p2_issue(ch, sl, poff=None):
        @pl.when(ch >= 2)
        def _():
            pltpu.make_async_copy(rows2[sl], R_sh.at[pl.ds(0, K2)],
                                  semR[sl]).wait()
        if poff is None:
            pltpu.sync_copy(idx_cat.at[pl.ds(tile_s + ch * K2, K2)], i_s2[sl])
            pltpu.sync_copy(idx_cat.at[pl.ds(tile_d + ch * K2, K2)], i_dq2[sl])
            for g in range(G2):
                dv = i_dq2[sl][pl.ds(g * 16, 16)]
                i_dq2[sl][pl.ds(g * 16, 16)] = dv + cid * N_NODES
        else:
            for g in range(G2):
                i_s2[sl][pl.ds(g * 16, 16)] = pairS[pl.ds(poff + g * 16, 16)]
                dv = pairD[pl.ds(poff + g * 16, 16)]
                i_dq2[sl][pl.ds(g * 16, 16)] = dv + cid * N_NODES
        pltpu.async_copy(tbuf.at[pl.ds(tile_t + ch * K2, K2)], tq2[sl], semt[sl])
        pltpu.async_copy(qbuf.at[i_dq2[sl]], rows2[sl], semq[sl])

    def _p2_finish(ch, sl):
        pltpu.make_async_copy(tbuf.at[pl.ds(tile_t + ch * K2, K2)], tq2[sl],
                              semt[sl]).wait()
        pltpu.make_async_copy(qbuf.at[pl.ds(0, K2)], rows2[sl], semq[sl]).wait()

        def _scale_grp(g, _):
            for j in range(16):
                i = g * 16 + j
                tv = tq2[sl][i, :]
                for s16 in range(HC // 16):
                    v = rows2[sl][i, pl.ds(s16 * 16, 16)]
                    rows2[sl][i, pl.ds(s16 * 16, 16)] = v * tv
            return 0
        lax.fori_loop(0, G2, _scale_grp, 0)
        pltpu.async_copy(rows2[sl], R_sh.at[i_s2[sl]], semR[sl], add=True)

    _p2_issue(0, 0)

    def _p2_pair(p, _):
        blk = (2 * p + 1) * K2
        pltpu.sync_copy(idx_cat.at[pl.ds(tile_s + blk, 2 * K2)], pairS)
        pltpu.sync_copy(idx_cat.at[pl.ds(tile_d + blk, 2 * K2)], pairD)
        _p2_issue(2 * p + 1, 1, 0)
        _p2_finish(2 * p, 0)
        _p2_issue(2 * p + 2, 0, K2)
        _p2_finish(2 * p + 1, 1)
        return 0
    lax.fori_loop(0, NCH2 // 2, _p2_pair, 0)
    _p2_finish(NCH2 - 1, 0)
    for sl in (0, 1):
        pltpu.make_async_copy(rows2[sl], R_sh.at[pl.ds(0, K2)],
                              semR[sl]).wait()
    plsc.subcore_barrier()

    def _r_out(s, _):
        off = nbase + s * SUB
        pltpu.sync_copy(R_sh.at[pl.ds(off, SUB)], rows0.at[pl.ds(0, SUB)])
        pltpu.sync_copy(rows0.at[pl.ds(0, SUB)],
                        rout.at[pl.ds(cid * N_NODES + off, SUB)])
        return 0
    lax.fori_loop(0, nsub, _r_out, 0)


def kernel(index, n, Z, W, b, a_l, a_r):
    num_nodes = Z.shape[0]
    zp2, tl2, trm2 = _dense_prep(Z, W, b, a_l, a_r)
    idx_cat = jnp.concatenate(
        [index.astype(jnp.int32).reshape(-1), jnp.zeros((K,), jnp.int32)])
    rout, _, _ = _sc_edges(idx_cat, tl2, trm2, zp2)
    rst = jnp.concatenate(
        [rout[:num_nodes].reshape(num_nodes, OUT_SIZE, HH),
         rout[num_nodes:].reshape(num_nodes, OUT_SIZE, HH)], axis=2)
    return rst * (jnp.asarray(n, dtype=rst.dtype) / num_nodes)
